# baseline (device time: 113161 ns/iter reference)
import jax
import jax.numpy as jnp
from jax import lax
from jax.experimental import pallas as pl
from jax.experimental.pallas import tpu as pltpu

B, S, HD_LOCAL, F = 2, 1024, 1024, 2048
S_HALF = S // 2


def kernel(O, Wo):
    O_flat = O.reshape(B, S, HD_LOCAL)

    def body(o_ref, wo_ref, out_ref, send_buf, recv_buf, send_sem, recv_sem):
        my_x = lax.axis_index("x")
        my_y = lax.axis_index("y")
        my_z = lax.axis_index("z")
        peer = (my_x, 1 - my_y, my_z)

        barrier_sem = pltpu.get_barrier_semaphore()
        pl.semaphore_signal(
            barrier_sem, inc=1, device_id=peer,
            device_id_type=pl.DeviceIdType.MESH,
        )
        pl.semaphore_wait(barrier_sem, 1)

        peer_start = (1 - my_y) * S_HALF
        my_start = my_y * S_HALF

        for b in range(B):
            send_buf[b, :, :] = jnp.dot(
                o_ref[b, pl.ds(peer_start, S_HALF)],
                wo_ref[...],
                preferred_element_type=jnp.float32,
            )

        rdma = pltpu.make_async_remote_copy(
            src_ref=send_buf,
            dst_ref=recv_buf,
            send_sem=send_sem,
            recv_sem=recv_sem,
            device_id=peer,
            device_id_type=pl.DeviceIdType.MESH,
        )
        rdma.start()

        for b in range(B):
            out_ref[b, :, :] = jnp.dot(
                o_ref[b, pl.ds(my_start, S_HALF)],
                wo_ref[...],
                preferred_element_type=jnp.float32,
            )

        rdma.wait()
        for b in range(B):
            out_ref[b, :, :] = out_ref[b, :, :] + recv_buf[b, :, :]

    return pl.pallas_call(
        body,
        out_shape=jax.ShapeDtypeStruct((B, S_HALF, F), jnp.float32),
        in_specs=[
            pl.BlockSpec(memory_space=pltpu.VMEM),
            pl.BlockSpec(memory_space=pltpu.VMEM),
        ],
        out_specs=pl.BlockSpec(memory_space=pltpu.VMEM),
        scratch_shapes=[
            pltpu.VMEM((B, S_HALF, F), jnp.float32),
            pltpu.VMEM((B, S_HALF, F), jnp.float32),
            pltpu.SemaphoreType.DMA,
            pltpu.SemaphoreType.DMA,
        ],
        compiler_params=pltpu.CompilerParams(collective_id=0),
    )(O_flat, Wo)


# device time: 109257 ns/iter; 1.0357x vs baseline; 1.0357x over previous
import jax
import jax.numpy as jnp
from jax import lax
from jax.experimental import pallas as pl
from jax.experimental.pallas import tpu as pltpu

B, S, HD_LOCAL, F = 2, 1024, 1024, 2048
S_HALF = S // 2
K = 8
R = S_HALF // K


def kernel(O, Wo):
    O_flat = O.reshape(B, S, HD_LOCAL)

    def body(o_ref, wo_ref, out_ref, send_buf, recv_buf, send_sems, recv_sems):
        my_x = lax.axis_index("x")
        my_y = lax.axis_index("y")
        my_z = lax.axis_index("z")
        peer = (my_x, 1 - my_y, my_z)

        barrier_sem = pltpu.get_barrier_semaphore()
        pl.semaphore_signal(
            barrier_sem, inc=1, device_id=peer,
            device_id_type=pl.DeviceIdType.MESH,
        )
        pl.semaphore_wait(barrier_sem, 1)

        peer_start = (1 - my_y) * S_HALF
        my_start = my_y * S_HALF

        def chunk_rdma(c):
            return pltpu.make_async_remote_copy(
                src_ref=send_buf.at[c],
                dst_ref=recv_buf.at[c],
                send_sem=send_sems.at[c],
                recv_sem=recv_sems.at[c],
                device_id=peer,
                device_id_type=pl.DeviceIdType.MESH,
            )

        for c in range(K):
            for b in range(B):
                send_buf[c, b, :, :] = jnp.dot(
                    o_ref[b, pl.ds(peer_start + c * R, R)],
                    wo_ref[...],
                    preferred_element_type=jnp.float32,
                )
            chunk_rdma(c).start()

        for b in range(B):
            out_ref[b, :, :] = jnp.dot(
                o_ref[b, pl.ds(my_start, S_HALF)],
                wo_ref[...],
                preferred_element_type=jnp.float32,
            )

        for c in range(K):
            rdma = chunk_rdma(c)
            rdma.wait_recv()
            for b in range(B):
                out_ref[b, pl.ds(c * R, R), :] = (
                    out_ref[b, pl.ds(c * R, R), :] + recv_buf[c, b, :, :]
                )
            rdma.wait_send()

    return pl.pallas_call(
        body,
        out_shape=jax.ShapeDtypeStruct((B, S_HALF, F), jnp.float32),
        in_specs=[
            pl.BlockSpec(memory_space=pltpu.VMEM),
            pl.BlockSpec(memory_space=pltpu.VMEM),
        ],
        out_specs=pl.BlockSpec(memory_space=pltpu.VMEM),
        scratch_shapes=[
            pltpu.VMEM((K, B, R, F), jnp.float32),
            pltpu.VMEM((K, B, R, F), jnp.float32),
            pltpu.SemaphoreType.DMA((K,)),
            pltpu.SemaphoreType.DMA((K,)),
        ],
        compiler_params=pltpu.CompilerParams(collective_id=0),
    )(O_flat, Wo)


# device time: 64293 ns/iter; 1.7601x vs baseline; 1.6994x over previous
import jax
import jax.numpy as jnp
from jax import lax
from jax.experimental import pallas as pl
from jax.experimental.pallas import tpu as pltpu

B, S, HD_LOCAL, F = 2, 1024, 1024, 2048
S_HALF = S // 2
K = 8
R = S_HALF // K


def kernel(O, Wo):
    O_flat = O.reshape(B, S, HD_LOCAL)

    def body(o_ref, wo_ref, out_ref, send_buf, recv_buf, send_sems, recv_sems):
        my_x = lax.axis_index("x")
        my_y = lax.axis_index("y")
        my_z = lax.axis_index("z")
        peer = (my_x, 1 - my_y, my_z)

        barrier_sem = pltpu.get_barrier_semaphore()
        pl.semaphore_signal(
            barrier_sem, inc=1, device_id=peer,
            device_id_type=pl.DeviceIdType.MESH,
        )
        pl.semaphore_wait(barrier_sem, 1)

        peer_start = (1 - my_y) * S_HALF
        my_start = my_y * S_HALF

        def chunk_rdma(c):
            return pltpu.make_async_remote_copy(
                src_ref=send_buf.at[c],
                dst_ref=recv_buf.at[c],
                send_sem=send_sems.at[c],
                recv_sem=recv_sems.at[c],
                device_id=peer,
                device_id_type=pl.DeviceIdType.MESH,
            )

        for c in range(K):
            for b in range(B):
                send_buf[c, b, :, :] = jnp.dot(
                    o_ref[b, pl.ds(peer_start + c * R, R)],
                    wo_ref[...],
                    preferred_element_type=jnp.float32,
                ).astype(jnp.bfloat16)
            chunk_rdma(c).start()

        for b in range(B):
            out_ref[b, :, :] = jnp.dot(
                o_ref[b, pl.ds(my_start, S_HALF)],
                wo_ref[...],
                preferred_element_type=jnp.float32,
            )

        for c in range(K):
            rdma = chunk_rdma(c)
            rdma.wait_recv()
            for b in range(B):
                out_ref[b, pl.ds(c * R, R), :] = (
                    out_ref[b, pl.ds(c * R, R), :]
                    + recv_buf[c, b, :, :].astype(jnp.float32)
                )
            rdma.wait_send()

    return pl.pallas_call(
        body,
        out_shape=jax.ShapeDtypeStruct((B, S_HALF, F), jnp.float32),
        in_specs=[
            pl.BlockSpec(memory_space=pltpu.VMEM),
            pl.BlockSpec(memory_space=pltpu.VMEM),
        ],
        out_specs=pl.BlockSpec(memory_space=pltpu.VMEM),
        scratch_shapes=[
            pltpu.VMEM((K, B, R, F), jnp.bfloat16),
            pltpu.VMEM((K, B, R, F), jnp.bfloat16),
            pltpu.SemaphoreType.DMA((K,)),
            pltpu.SemaphoreType.DMA((K,)),
        ],
        compiler_params=pltpu.CompilerParams(collective_id=0),
    )(O_flat, Wo)


# device time: 56715 ns/iter; 1.9953x vs baseline; 1.1336x over previous
import jax
import jax.numpy as jnp
from jax import lax
from jax.experimental import pallas as pl
from jax.experimental.pallas import tpu as pltpu

B, S, HD_LOCAL, F = 2, 1024, 1024, 2048
S_HALF = S // 2
Q = S_HALF // 4


def kernel(O, Wo):
    O_flat = O.reshape(B, S, HD_LOCAL)

    def body(o_ref, wo_ref, out_ref, psend, precv, agsend, agrecv,
             y_send_sem, y_recv_sem, ag_send_sems, ag_recv_sems):
        my_x = lax.axis_index("x")
        my_y = lax.axis_index("y")
        my_z = lax.axis_index("z")
        y_peer = (my_x, 1 - my_y, my_z)
        x_nbr = (1 - my_x, my_y, my_z)
        z_nbr = (my_x, my_y, 1 - my_z)
        diag = (1 - my_x, my_y, 1 - my_z)
        r_idx = 2 * my_x + my_z
        x_nbr_r = 2 * (1 - my_x) + my_z
        z_nbr_r = 2 * my_x + (1 - my_z)
        diag_r = 2 * (1 - my_x) + (1 - my_z)

        barrier_sem = pltpu.get_barrier_semaphore()
        for nbr in (y_peer, x_nbr, z_nbr, diag):
            pl.semaphore_signal(
                barrier_sem, inc=1, device_id=nbr,
                device_id_type=pl.DeviceIdType.MESH,
            )
        pl.semaphore_wait(barrier_sem, 4)

        my_start = my_y * S_HALF + r_idx * Q
        peer_start = (1 - my_y) * S_HALF + r_idx * Q

        for b in range(B):
            psend[b, :, :] = jnp.dot(
                o_ref[b, pl.ds(peer_start, Q)],
                wo_ref[...],
                preferred_element_type=jnp.float32,
            ).astype(jnp.bfloat16)
        y_rdma = pltpu.make_async_remote_copy(
            src_ref=psend, dst_ref=precv,
            send_sem=y_send_sem, recv_sem=y_recv_sem,
            device_id=y_peer, device_id_type=pl.DeviceIdType.MESH,
        )
        y_rdma.start()

        for b in range(B):
            out_ref[b, pl.ds(r_idx * Q, Q), :] = jnp.dot(
                o_ref[b, pl.ds(my_start, Q)],
                wo_ref[...],
                preferred_element_type=jnp.float32,
            )

        y_rdma.wait_recv()
        for b in range(B):
            q = out_ref[b, pl.ds(r_idx * Q, Q), :] + precv[b].astype(jnp.float32)
            out_ref[b, pl.ds(r_idx * Q, Q), :] = q
            agsend[b, :, :] = q.astype(jnp.bfloat16)

        sends = []
        for i, nbr in enumerate((x_nbr, z_nbr, diag)):
            s = pltpu.make_async_remote_copy(
                src_ref=agsend, dst_ref=agrecv.at[r_idx],
                send_sem=ag_send_sems.at[i], recv_sem=ag_recv_sems.at[r_idx],
                device_id=nbr, device_id_type=pl.DeviceIdType.MESH,
            )
            s.start()
            sends.append(s)

        for src_r in (x_nbr_r, z_nbr_r, diag_r):
            recv = pltpu.make_async_remote_copy(
                src_ref=agsend, dst_ref=agrecv.at[src_r],
                send_sem=ag_send_sems.at[0], recv_sem=ag_recv_sems.at[src_r],
                device_id=y_peer, device_id_type=pl.DeviceIdType.MESH,
            )
            recv.wait_recv()
            for b in range(B):
                out_ref[b, pl.ds(src_r * Q, Q), :] = (
                    agrecv[src_r, b, :, :].astype(jnp.float32)
                )

        y_rdma.wait_send()
        for s in sends:
            s.wait_send()

    return pl.pallas_call(
        body,
        out_shape=jax.ShapeDtypeStruct((B, S_HALF, F), jnp.float32),
        in_specs=[
            pl.BlockSpec(memory_space=pltpu.VMEM),
            pl.BlockSpec(memory_space=pltpu.VMEM),
        ],
        out_specs=pl.BlockSpec(memory_space=pltpu.VMEM),
        scratch_shapes=[
            pltpu.VMEM((B, Q, F), jnp.bfloat16),
            pltpu.VMEM((B, Q, F), jnp.bfloat16),
            pltpu.VMEM((B, Q, F), jnp.bfloat16),
            pltpu.VMEM((4, B, Q, F), jnp.bfloat16),
            pltpu.SemaphoreType.DMA,
            pltpu.SemaphoreType.DMA,
            pltpu.SemaphoreType.DMA((3,)),
            pltpu.SemaphoreType.DMA((4,)),
        ],
        compiler_params=pltpu.CompilerParams(collective_id=0),
    )(O_flat, Wo)


# device time: 50128 ns/iter; 2.2574x vs baseline; 1.1314x over previous
import jax
import jax.numpy as jnp
from jax import lax
from jax.experimental import pallas as pl
from jax.experimental.pallas import tpu as pltpu

B, S, HD_LOCAL, F = 2, 1024, 1024, 2048
S_HALF = S // 2
Q = S_HALF // 4
NSUB = 4
QS = Q // NSUB


def kernel(O, Wo):
    O_flat = O.reshape(B, S, HD_LOCAL)

    def body(o_ref, wo_ref, out_ref, psend, precv, agsend, agrecv,
             y_send_sems, y_recv_sems, ag_send_sems, ag_recv_sems):
        my_x = lax.axis_index("x")
        my_y = lax.axis_index("y")
        my_z = lax.axis_index("z")
        y_peer = (my_x, 1 - my_y, my_z)
        x_nbr = (1 - my_x, my_y, my_z)
        z_nbr = (my_x, my_y, 1 - my_z)
        diag = (1 - my_x, my_y, 1 - my_z)
        r_idx = 2 * my_x + my_z
        x_nbr_r = 2 * (1 - my_x) + my_z
        z_nbr_r = 2 * my_x + (1 - my_z)
        diag_r = 2 * (1 - my_x) + (1 - my_z)

        barrier_sem = pltpu.get_barrier_semaphore()
        for nbr in (y_peer, x_nbr, z_nbr, diag):
            pl.semaphore_signal(
                barrier_sem, inc=1, device_id=nbr,
                device_id_type=pl.DeviceIdType.MESH,
            )
        pl.semaphore_wait(barrier_sem, 4)

        my_start = my_y * S_HALF + r_idx * Q
        peer_start = (1 - my_y) * S_HALF + r_idx * Q

        def y_rdma(s):
            return pltpu.make_async_remote_copy(
                src_ref=psend.at[s], dst_ref=precv.at[s],
                send_sem=y_send_sems.at[s], recv_sem=y_recv_sems.at[s],
                device_id=y_peer, device_id_type=pl.DeviceIdType.MESH,
            )

        for s in range(NSUB):
            for b in range(B):
                psend[s, b, :, :] = jnp.dot(
                    o_ref[b, pl.ds(peer_start + s * QS, QS)],
                    wo_ref[...],
                    preferred_element_type=jnp.float32,
                ).astype(jnp.bfloat16)
            y_rdma(s).start()

        for b in range(B):
            out_ref[b, pl.ds(r_idx * Q, Q), :] = jnp.dot(
                o_ref[b, pl.ds(my_start, Q)],
                wo_ref[...],
                preferred_element_type=jnp.float32,
            )

        ag_sends = []
        for s in range(NSUB):
            y_rdma(s).wait_recv()
            for b in range(B):
                row0 = r_idx * Q + s * QS
                q = (out_ref[b, pl.ds(row0, QS), :]
                     + precv[s, b].astype(jnp.float32))
                out_ref[b, pl.ds(row0, QS), :] = q
                agsend[s, b, :, :] = q.astype(jnp.bfloat16)
            for i, nbr in enumerate((x_nbr, z_nbr, diag)):
                snd = pltpu.make_async_remote_copy(
                    src_ref=agsend.at[s], dst_ref=agrecv.at[r_idx, s],
                    send_sem=ag_send_sems.at[i, s],
                    recv_sem=ag_recv_sems.at[r_idx, s],
                    device_id=nbr, device_id_type=pl.DeviceIdType.MESH,
                )
                snd.start()
                ag_sends.append(snd)

        for src_r in (x_nbr_r, z_nbr_r, diag_r):
            for s in range(NSUB):
                recv = pltpu.make_async_remote_copy(
                    src_ref=agsend.at[s], dst_ref=agrecv.at[src_r, s],
                    send_sem=ag_send_sems.at[0, s],
                    recv_sem=ag_recv_sems.at[src_r, s],
                    device_id=y_peer, device_id_type=pl.DeviceIdType.MESH,
                )
                recv.wait_recv()
                for b in range(B):
                    out_ref[b, pl.ds(src_r * Q + s * QS, QS), :] = (
                        agrecv[src_r, s, b, :, :].astype(jnp.float32)
                    )

        for s in range(NSUB):
            y_rdma(s).wait_send()
        for snd in ag_sends:
            snd.wait_send()

    return pl.pallas_call(
        body,
        out_shape=jax.ShapeDtypeStruct((B, S_HALF, F), jnp.float32),
        in_specs=[
            pl.BlockSpec(memory_space=pltpu.VMEM),
            pl.BlockSpec(memory_space=pltpu.VMEM),
        ],
        out_specs=pl.BlockSpec(memory_space=pltpu.VMEM),
        scratch_shapes=[
            pltpu.VMEM((NSUB, B, QS, F), jnp.bfloat16),
            pltpu.VMEM((NSUB, B, QS, F), jnp.bfloat16),
            pltpu.VMEM((NSUB, B, QS, F), jnp.bfloat16),
            pltpu.VMEM((4, NSUB, B, QS, F), jnp.bfloat16),
            pltpu.SemaphoreType.DMA((NSUB,)),
            pltpu.SemaphoreType.DMA((NSUB,)),
            pltpu.SemaphoreType.DMA((3, NSUB)),
            pltpu.SemaphoreType.DMA((4, NSUB)),
        ],
        compiler_params=pltpu.CompilerParams(collective_id=0),
    )(O_flat, Wo)
